# baseline (speedup 1.0000x reference)
import functools

import jax
import jax.numpy as jnp
from jax import lax
from jax.experimental import pallas as pl
from jax.experimental.pallas import tpu as pltpu
from jax.experimental.pallas import tpu_sc as plsc

_INFO = plsc.get_sparse_core_info()
_NC = _INFO.num_cores
_NS = _INFO.num_subcores
_NW = _NC * _NS
_C = 64


_R = 6


def _make_prop(N, D, E):
    n_chunks = E // _C
    assert n_chunks * _C == E
    rs = (N // _NS) // 8 * 8
    tail = N - rs * _NS
    nbase, extra = divmod(n_chunks, _NW)
    assert nbase % _R == 0 and nbase >= 2 * _R
    mesh = plsc.VectorSubcoreMesh(core_axis_name="c", subcore_axis_name="s")

    @functools.partial(
        pl.kernel,
        mesh=mesh,
        out_type=jax.ShapeDtypeStruct((_NC, N, D), jnp.float32),
        scratch_types=(
            [pltpu.VMEM((2, _C), jnp.int32)] * _R
            + [pltpu.VMEM((_C, D), jnp.float32)] * _R
            + [pltpu.VMEM_SHARED((N, D), jnp.float32)]
            + [pltpu.SemaphoreType.DMA] * (2 * _R)
        ),
    )
    def prop(x_hbm, e3_hbm, zeros_hbm, out_hbm, *scr):
        sd = scr[:_R]
        rb = scr[_R:2 * _R]
        acc = scr[2 * _R]
        gs = scr[2 * _R + 1:3 * _R + 1]
        ss = scr[3 * _R + 1:4 * _R + 1]
        cid = lax.axis_index("c")
        sid = lax.axis_index("s")
        wid = sid * _NC + cid
        r0 = sid * rs
        pltpu.sync_copy(zeros_hbm.at[pl.ds(r0, rs)], acc.at[pl.ds(r0, rs)])
        if tail:
            @pl.when(sid == _NS - 1)
            def _():
                pltpu.sync_copy(zeros_hbm.at[pl.ds(rs * _NS, tail)],
                                acc.at[pl.ds(rs * _NS, tail)])
        plsc.subcore_barrier()

        def fetch(chunk, b):
            pltpu.sync_copy(e3_hbm.at[chunk], sd[b])
            pltpu.async_copy(x_hbm.at[sd[b].at[0]], rb[b], gs[b])

        for b in range(_R - 1):
            fetch(wid + b * _NW, b)

        def body(j, carry):
            for k in range(_R):
                s = _R * j + k
                q = (k + _R - 1) % _R
                pltpu.make_async_copy(x_hbm.at[sd[k].at[0]], rb[k],
                                      gs[k]).wait()
                pltpu.async_copy(rb[k], acc.at[sd[k].at[1]], ss[k], add=True)

                @pl.when(s >= 1)
                def _():
                    pltpu.make_async_copy(rb[q], acc.at[sd[q].at[1]],
                                          ss[q]).wait()

                @pl.when(s + _R - 1 < nbase)
                def _():
                    fetch(wid + (s + _R - 1) * _NW, q)
            return carry

        lax.fori_loop(0, nbase // _R, body, 0)
        pltpu.make_async_copy(rb[_R - 1], acc.at[sd[_R - 1].at[1]],
                              ss[_R - 1]).wait()

        if extra:
            @pl.when(wid < extra)
            def _():
                pltpu.sync_copy(e3_hbm.at[nbase * _NW + wid], sd[0])
                pltpu.async_copy(x_hbm.at[sd[0].at[0]], rb[0], gs[0]).wait()
                pltpu.sync_copy(rb[0], acc.at[sd[0].at[1]], add=True)

        plsc.subcore_barrier()
        pltpu.sync_copy(acc.at[pl.ds(r0, rs)], out_hbm.at[cid, pl.ds(r0, rs)])
        if tail:
            @pl.when(sid == _NS - 1)
            def _():
                pltpu.sync_copy(acc.at[pl.ds(rs * _NS, tail)],
                                out_hbm.at[cid, pl.ds(rs * _NS, tail)])

    return prop


def _make_deg(N, E, W=128):
    n_chunks = E // _C
    rs = (N // _NS) // 8 * 8
    tail = N - rs * _NS
    nbase, extra = divmod(n_chunks, _NW)
    assert nbase % 2 == 0 and nbase >= 4
    mesh = plsc.VectorSubcoreMesh(core_axis_name="c", subcore_axis_name="s")

    @functools.partial(
        pl.kernel,
        mesh=mesh,
        out_type=jax.ShapeDtypeStruct((_NC, N, W), jnp.float32),
        scratch_types=[
            pltpu.VMEM((_C,), jnp.int32),
            pltpu.VMEM((_C,), jnp.int32),
            pltpu.VMEM((_C, W), jnp.float32),
            pltpu.VMEM_SHARED((N, W), jnp.float32),
            pltpu.SemaphoreType.DMA,
            pltpu.SemaphoreType.DMA,
        ],
    )
    def deg(src_hbm, ones_hbm, zeros_hbm, out_hbm,
            idx0, idx1, ones_v, acc, ss0, ss1):
        cid = lax.axis_index("c")
        sid = lax.axis_index("s")
        wid = sid * _NC + cid
        r0 = sid * rs
        pltpu.sync_copy(zeros_hbm.at[pl.ds(r0, rs)], acc.at[pl.ds(r0, rs)])
        if tail:
            @pl.when(sid == _NS - 1)
            def _():
                pltpu.sync_copy(zeros_hbm.at[pl.ds(rs * _NS, tail)],
                                acc.at[pl.ds(rs * _NS, tail)])
        pltpu.sync_copy(ones_hbm, ones_v)
        plsc.subcore_barrier()

        pltpu.sync_copy(src_hbm.at[pl.ds(wid * _C, _C)], idx0)

        def body(j, carry):
            pltpu.async_copy(ones_v, acc.at[idx0], ss0, add=True)

            @pl.when(j >= 1)
            def _():
                pltpu.make_async_copy(ones_v, acc.at[idx1], ss1).wait()

            pltpu.sync_copy(
                src_hbm.at[pl.ds((wid + (2 * j + 1) * _NW) * _C, _C)], idx1)
            pltpu.async_copy(ones_v, acc.at[idx1], ss1, add=True)

            @pl.when(j < nbase // 2 - 1)
            def _():
                pltpu.make_async_copy(ones_v, acc.at[idx0], ss0).wait()
                pltpu.sync_copy(
                    src_hbm.at[pl.ds((wid + (2 * j + 2) * _NW) * _C, _C)],
                    idx0)
            return carry

        lax.fori_loop(0, nbase // 2, body, 0)
        pltpu.make_async_copy(ones_v, acc.at[idx0], ss0).wait()
        pltpu.make_async_copy(ones_v, acc.at[idx1], ss1).wait()

        if extra:
            @pl.when(wid < extra)
            def _():
                pltpu.sync_copy(
                    src_hbm.at[pl.ds((nbase * _NW + wid) * _C, _C)], idx0)
                pltpu.sync_copy(ones_v, acc.at[idx0], add=True)

        plsc.subcore_barrier()
        pltpu.sync_copy(acc.at[pl.ds(r0, rs)], out_hbm.at[cid, pl.ds(r0, rs)])
        if tail:
            @pl.when(sid == _NS - 1)
            def _():
                pltpu.sync_copy(acc.at[pl.ds(rs * _NS, tail)],
                                out_hbm.at[cid, pl.ds(rs * _NS, tail)])

    return deg


_B = 1000


def _dinv_block(deg_ref):
    deg = deg_ref[0, :, 0:1] + deg_ref[1, :, 0:1]
    return jnp.where(deg > 0, lax.rsqrt(jnp.maximum(deg, 1e-12)), 0.0)


def _tc_pre(degp, x):
    N, D = x.shape

    def body(deg_ref, x_ref, o_ref):
        o_ref[...] = _dinv_block(deg_ref) * x_ref[...]

    return pl.pallas_call(
        body,
        grid=(N // _B,),
        in_specs=[
            pl.BlockSpec((2, _B, 128), lambda i: (0, i, 0)),
            pl.BlockSpec((_B, D), lambda i: (i, 0)),
        ],
        out_specs=pl.BlockSpec((_B, D), lambda i: (i, 0)),
        out_shape=jax.ShapeDtypeStruct((N, D), jnp.float32),
    )(degp, x)


def _tc_mid(degp, s1, x, W):
    N, D = x.shape

    def body(deg_ref, s_ref, x_ref, w_ref, txp_ref, acc_ref):
        dinv = _dinv_block(deg_ref)
        tx1 = -dinv * (s_ref[0] + s_ref[1])
        txp_ref[...] = dinv * tx1
        acc_ref[...] = (
            jnp.dot(x_ref[...], w_ref[0], preferred_element_type=jnp.float32)
            + jnp.dot(tx1, w_ref[1], preferred_element_type=jnp.float32))

    return pl.pallas_call(
        body,
        grid=(N // _B,),
        in_specs=[
            pl.BlockSpec((2, _B, 128), lambda i: (0, i, 0)),
            pl.BlockSpec((2, _B, D), lambda i: (0, i, 0)),
            pl.BlockSpec((_B, D), lambda i: (i, 0)),
            pl.BlockSpec((3, D, D), lambda i: (0, 0, 0)),
        ],
        out_specs=[
            pl.BlockSpec((_B, D), lambda i: (i, 0)),
            pl.BlockSpec((_B, D), lambda i: (i, 0)),
        ],
        out_shape=[
            jax.ShapeDtypeStruct((N, D), jnp.float32),
            jax.ShapeDtypeStruct((N, D), jnp.float32),
        ],
    )(degp, s1, x, W)


def _tc_fin(degp, s2, x, acc, W, b, relu):
    N, D = x.shape

    def body(deg_ref, s_ref, x_ref, acc_ref, w_ref, b_ref, *outs):
        dinv = _dinv_block(deg_ref)
        tx2 = -2.0 * dinv * (s_ref[0] + s_ref[1]) - x_ref[...]
        o = (acc_ref[...]
             + jnp.dot(tx2, w_ref[2], preferred_element_type=jnp.float32)
             + b_ref[...])
        if relu:
            h = jnp.maximum(o, 0.0)
            outs[0][...] = h
            outs[1][...] = dinv * h
        else:
            outs[0][...] = o

    n_out = 2 if relu else 1
    return pl.pallas_call(
        body,
        grid=(N // _B,),
        in_specs=[
            pl.BlockSpec((2, _B, 128), lambda i: (0, i, 0)),
            pl.BlockSpec((2, _B, D), lambda i: (0, i, 0)),
            pl.BlockSpec((_B, D), lambda i: (i, 0)),
            pl.BlockSpec((_B, D), lambda i: (i, 0)),
            pl.BlockSpec((3, D, D), lambda i: (0, 0, 0)),
            pl.BlockSpec((1, D), lambda i: (0, 0)),
        ],
        out_specs=[pl.BlockSpec((_B, D), lambda i: (i, 0))] * n_out,
        out_shape=[jax.ShapeDtypeStruct((N, D), jnp.float32)] * n_out,
    )(degp, s2, x, acc, W, b)


def kernel(x, edge_index, W1, b1, W2, b2):
    N, D = x.shape
    E = edge_index.shape[1]
    src = edge_index[0]
    e3 = edge_index.reshape(2, E // _C, _C).transpose(1, 0, 2)

    zeros_nd = jnp.zeros((N, D), jnp.float32)
    ones_cd = jnp.ones((_C, D), jnp.float32)
    b1r = b1.reshape(1, D)
    b2r = b2.reshape(1, D)

    prop = _make_prop(N, D, E)
    degf = _make_deg(N, E)

    degp = degf(src, ones_cd, zeros_nd)

    xp = _tc_pre(degp, x)
    s1 = prop(xp, e3, zeros_nd)
    tx1p, acc1 = _tc_mid(degp, s1, x, W1)
    s2 = prop(tx1p, e3, zeros_nd)
    h, hp = _tc_fin(degp, s2, x, acc1, W1, b1r, relu=True)

    s3 = prop(hp, e3, zeros_nd)
    tx1p2, acc2 = _tc_mid(degp, s3, h, W2)
    s4 = prop(tx1p2, e3, zeros_nd)
    (out,) = _tc_fin(degp, s4, h, acc2, W2, b2r, relu=False)
    return out


# speedup vs baseline: 1.0514x; 1.0514x over previous
import functools

import jax
import jax.numpy as jnp
from jax import lax
from jax.experimental import pallas as pl
from jax.experimental.pallas import tpu as pltpu
from jax.experimental.pallas import tpu_sc as plsc

_INFO = plsc.get_sparse_core_info()
_NC = _INFO.num_cores
_NS = _INFO.num_subcores
_NW = _NC * _NS
_C = 128


_R = 3


def _make_prop(N, D, E):
    n_chunks = E // _C
    assert n_chunks * _C == E
    rs = (N // _NS) // 8 * 8
    tail = N - rs * _NS
    nbase, extra = divmod(n_chunks, _NW)
    assert nbase % _R == 0 and nbase >= 2 * _R
    mesh = plsc.VectorSubcoreMesh(core_axis_name="c", subcore_axis_name="s")

    @functools.partial(
        pl.kernel,
        mesh=mesh,
        out_type=jax.ShapeDtypeStruct((_NC, N, D), jnp.float32),
        scratch_types=(
            [pltpu.VMEM((2, _C), jnp.int32)] * _R
            + [pltpu.VMEM((_C, D), jnp.float32)] * _R
            + [pltpu.VMEM_SHARED((N, D), jnp.float32)]
            + [pltpu.SemaphoreType.DMA] * (2 * _R)
        ),
    )
    def prop(x_hbm, e3_hbm, zeros_hbm, out_hbm, *scr):
        sd = scr[:_R]
        rb = scr[_R:2 * _R]
        acc = scr[2 * _R]
        gs = scr[2 * _R + 1:3 * _R + 1]
        ss = scr[3 * _R + 1:4 * _R + 1]
        cid = lax.axis_index("c")
        sid = lax.axis_index("s")
        wid = sid * _NC + cid
        r0 = sid * rs
        pltpu.sync_copy(zeros_hbm.at[pl.ds(r0, rs)], acc.at[pl.ds(r0, rs)])
        if tail:
            @pl.when(sid == _NS - 1)
            def _():
                pltpu.sync_copy(zeros_hbm.at[pl.ds(rs * _NS, tail)],
                                acc.at[pl.ds(rs * _NS, tail)])
        plsc.subcore_barrier()

        def fetch(chunk, b):
            pltpu.sync_copy(e3_hbm.at[chunk], sd[b])
            pltpu.async_copy(x_hbm.at[sd[b].at[0]], rb[b], gs[b])

        for b in range(_R - 1):
            fetch(wid + b * _NW, b)

        def body(j, carry):
            for k in range(_R):
                s = _R * j + k
                q = (k + _R - 1) % _R
                pltpu.make_async_copy(x_hbm.at[sd[k].at[0]], rb[k],
                                      gs[k]).wait()
                pltpu.async_copy(rb[k], acc.at[sd[k].at[1]], ss[k], add=True)

                @pl.when(s >= 1)
                def _():
                    pltpu.make_async_copy(rb[q], acc.at[sd[q].at[1]],
                                          ss[q]).wait()

                @pl.when(s + _R - 1 < nbase)
                def _():
                    fetch(wid + (s + _R - 1) * _NW, q)
            return carry

        lax.fori_loop(0, nbase // _R, body, 0)
        pltpu.make_async_copy(rb[_R - 1], acc.at[sd[_R - 1].at[1]],
                              ss[_R - 1]).wait()

        if extra:
            @pl.when(wid < extra)
            def _():
                pltpu.sync_copy(e3_hbm.at[nbase * _NW + wid], sd[0])
                pltpu.async_copy(x_hbm.at[sd[0].at[0]], rb[0], gs[0]).wait()
                pltpu.sync_copy(rb[0], acc.at[sd[0].at[1]], add=True)

        plsc.subcore_barrier()
        pltpu.sync_copy(acc.at[pl.ds(r0, rs)], out_hbm.at[cid, pl.ds(r0, rs)])
        if tail:
            @pl.when(sid == _NS - 1)
            def _():
                pltpu.sync_copy(acc.at[pl.ds(rs * _NS, tail)],
                                out_hbm.at[cid, pl.ds(rs * _NS, tail)])

    return prop


def _make_deg(N, E, W=128):
    n_chunks = E // _C
    rs = (N // _NS) // 8 * 8
    tail = N - rs * _NS
    nbase, extra = divmod(n_chunks, _NW)
    assert nbase % 2 == 0 and nbase >= 4
    mesh = plsc.VectorSubcoreMesh(core_axis_name="c", subcore_axis_name="s")

    @functools.partial(
        pl.kernel,
        mesh=mesh,
        out_type=jax.ShapeDtypeStruct((_NC, N, W), jnp.float32),
        scratch_types=[
            pltpu.VMEM((_C,), jnp.int32),
            pltpu.VMEM((_C,), jnp.int32),
            pltpu.VMEM((_C, W), jnp.float32),
            pltpu.VMEM_SHARED((N, W), jnp.float32),
            pltpu.SemaphoreType.DMA,
            pltpu.SemaphoreType.DMA,
        ],
    )
    def deg(src_hbm, ones_hbm, zeros_hbm, out_hbm,
            idx0, idx1, ones_v, acc, ss0, ss1):
        cid = lax.axis_index("c")
        sid = lax.axis_index("s")
        wid = sid * _NC + cid
        r0 = sid * rs
        pltpu.sync_copy(zeros_hbm.at[pl.ds(r0, rs)], acc.at[pl.ds(r0, rs)])
        if tail:
            @pl.when(sid == _NS - 1)
            def _():
                pltpu.sync_copy(zeros_hbm.at[pl.ds(rs * _NS, tail)],
                                acc.at[pl.ds(rs * _NS, tail)])
        pltpu.sync_copy(ones_hbm, ones_v)
        plsc.subcore_barrier()

        pltpu.sync_copy(src_hbm.at[pl.ds(wid * _C, _C)], idx0)

        def body(j, carry):
            pltpu.async_copy(ones_v, acc.at[idx0], ss0, add=True)

            @pl.when(j >= 1)
            def _():
                pltpu.make_async_copy(ones_v, acc.at[idx1], ss1).wait()

            pltpu.sync_copy(
                src_hbm.at[pl.ds((wid + (2 * j + 1) * _NW) * _C, _C)], idx1)
            pltpu.async_copy(ones_v, acc.at[idx1], ss1, add=True)

            @pl.when(j < nbase // 2 - 1)
            def _():
                pltpu.make_async_copy(ones_v, acc.at[idx0], ss0).wait()
                pltpu.sync_copy(
                    src_hbm.at[pl.ds((wid + (2 * j + 2) * _NW) * _C, _C)],
                    idx0)
            return carry

        lax.fori_loop(0, nbase // 2, body, 0)
        pltpu.make_async_copy(ones_v, acc.at[idx0], ss0).wait()
        pltpu.make_async_copy(ones_v, acc.at[idx1], ss1).wait()

        if extra:
            @pl.when(wid < extra)
            def _():
                pltpu.sync_copy(
                    src_hbm.at[pl.ds((nbase * _NW + wid) * _C, _C)], idx0)
                pltpu.sync_copy(ones_v, acc.at[idx0], add=True)

        plsc.subcore_barrier()
        pltpu.sync_copy(acc.at[pl.ds(r0, rs)], out_hbm.at[cid, pl.ds(r0, rs)])
        if tail:
            @pl.when(sid == _NS - 1)
            def _():
                pltpu.sync_copy(acc.at[pl.ds(rs * _NS, tail)],
                                out_hbm.at[cid, pl.ds(rs * _NS, tail)])

    return deg


_B = 1000


def _tc_pre(degp, x):
    N, D = x.shape

    def body(deg_ref, x_ref, o_ref, dn_ref):
        deg = deg_ref[0, :, 0:1] + deg_ref[1, :, 0:1]
        dinv = jnp.where(deg > 0, lax.rsqrt(jnp.maximum(deg, 1e-12)), 0.0)
        o_ref[...] = dinv * x_ref[...]
        dn_ref[...] = jnp.broadcast_to(dinv, (dinv.shape[0], 8))

    return pl.pallas_call(
        body,
        grid=(N // _B,),
        in_specs=[
            pl.BlockSpec((2, _B, 128), lambda i: (0, i, 0)),
            pl.BlockSpec((_B, D), lambda i: (i, 0)),
        ],
        out_specs=[
            pl.BlockSpec((_B, D), lambda i: (i, 0)),
            pl.BlockSpec((_B, 8), lambda i: (i, 0)),
        ],
        out_shape=[
            jax.ShapeDtypeStruct((N, D), jnp.float32),
            jax.ShapeDtypeStruct((N, 8), jnp.float32),
        ],
    )(degp, x)


def _tc_mid(dn, s1, x, W):
    N, D = x.shape

    def body(dn_ref, s_ref, x_ref, w_ref, txp_ref, acc_ref):
        dinv = dn_ref[:, 0:1]
        tx1 = -dinv * (s_ref[0] + s_ref[1])
        txp_ref[...] = dinv * tx1
        acc_ref[...] = (
            jnp.dot(x_ref[...], w_ref[0], preferred_element_type=jnp.float32)
            + jnp.dot(tx1, w_ref[1], preferred_element_type=jnp.float32))

    return pl.pallas_call(
        body,
        grid=(N // _B,),
        in_specs=[
            pl.BlockSpec((_B, 8), lambda i: (i, 0)),
            pl.BlockSpec((2, _B, D), lambda i: (0, i, 0)),
            pl.BlockSpec((_B, D), lambda i: (i, 0)),
            pl.BlockSpec((3, D, D), lambda i: (0, 0, 0)),
        ],
        out_specs=[
            pl.BlockSpec((_B, D), lambda i: (i, 0)),
            pl.BlockSpec((_B, D), lambda i: (i, 0)),
        ],
        out_shape=[
            jax.ShapeDtypeStruct((N, D), jnp.float32),
            jax.ShapeDtypeStruct((N, D), jnp.float32),
        ],
    )(dn, s1, x, W)


def _tc_fin(dn, s2, x, acc, W, b, relu):
    N, D = x.shape

    def body(dn_ref, s_ref, x_ref, acc_ref, w_ref, b_ref, *outs):
        dinv = dn_ref[:, 0:1]
        tx2 = -2.0 * dinv * (s_ref[0] + s_ref[1]) - x_ref[...]
        o = (acc_ref[...]
             + jnp.dot(tx2, w_ref[2], preferred_element_type=jnp.float32)
             + b_ref[...])
        if relu:
            h = jnp.maximum(o, 0.0)
            outs[0][...] = h
            outs[1][...] = dinv * h
        else:
            outs[0][...] = o

    n_out = 2 if relu else 1
    return pl.pallas_call(
        body,
        grid=(N // _B,),
        in_specs=[
            pl.BlockSpec((_B, 8), lambda i: (i, 0)),
            pl.BlockSpec((2, _B, D), lambda i: (0, i, 0)),
            pl.BlockSpec((_B, D), lambda i: (i, 0)),
            pl.BlockSpec((_B, D), lambda i: (i, 0)),
            pl.BlockSpec((3, D, D), lambda i: (0, 0, 0)),
            pl.BlockSpec((1, D), lambda i: (0, 0)),
        ],
        out_specs=[pl.BlockSpec((_B, D), lambda i: (i, 0))] * n_out,
        out_shape=[jax.ShapeDtypeStruct((N, D), jnp.float32)] * n_out,
    )(dn, s2, x, acc, W, b)


def kernel(x, edge_index, W1, b1, W2, b2):
    N, D = x.shape
    E = edge_index.shape[1]
    src = edge_index[0]
    e3 = edge_index.reshape(2, E // _C, _C).transpose(1, 0, 2)

    zeros_nd = jnp.zeros((N, D), jnp.float32)
    ones_cd = jnp.ones((_C, D), jnp.float32)
    b1r = b1.reshape(1, D)
    b2r = b2.reshape(1, D)

    prop = _make_prop(N, D, E)
    degf = _make_deg(N, E)

    degp = degf(src, ones_cd, zeros_nd)

    xp, dn = _tc_pre(degp, x)
    s1 = prop(xp, e3, zeros_nd)
    tx1p, acc1 = _tc_mid(dn, s1, x, W1)
    s2 = prop(tx1p, e3, zeros_nd)
    h, hp = _tc_fin(dn, s2, x, acc1, W1, b1r, relu=True)

    s3 = prop(hp, e3, zeros_nd)
    tx1p2, acc2 = _tc_mid(dn, s3, h, W2)
    s4 = prop(tx1p2, e3, zeros_nd)
    (out,) = _tc_fin(dn, s4, h, acc2, W2, b2r, relu=False)
    return out


# speedup vs baseline: 1.0561x; 1.0044x over previous
import functools

import jax
import jax.numpy as jnp
from jax import lax
from jax.experimental import pallas as pl
from jax.experimental.pallas import tpu as pltpu
from jax.experimental.pallas import tpu_sc as plsc

_INFO = plsc.get_sparse_core_info()
_NC = _INFO.num_cores
_NS = _INFO.num_subcores
_NW = _NC * _NS
_C = 128


_R = 3


def _make_prop(N, D, E):
    n_chunks = E // _C
    assert n_chunks * _C == E
    rs = (N // _NS) // 8 * 8
    tail = N - rs * _NS
    nbase, extra = divmod(n_chunks, _NW)
    assert nbase % _R == 0 and nbase >= 2 * _R
    mesh = plsc.VectorSubcoreMesh(core_axis_name="c", subcore_axis_name="s")

    @functools.partial(
        pl.kernel,
        mesh=mesh,
        out_type=jax.ShapeDtypeStruct((_NC, N, D), jnp.float32),
        scratch_types=(
            [pltpu.VMEM((2, _C), jnp.int32)] * _R
            + [pltpu.VMEM((_C, D), jnp.float32)] * _R
            + [pltpu.VMEM_SHARED((N, D), jnp.float32)]
            + [pltpu.SemaphoreType.DMA] * (2 * _R)
        ),
    )
    def prop(x_hbm, e3_hbm, zeros_hbm, out_hbm, *scr):
        sd = scr[:_R]
        rb = scr[_R:2 * _R]
        acc = scr[2 * _R]
        gs = scr[2 * _R + 1:3 * _R + 1]
        ss = scr[3 * _R + 1:4 * _R + 1]
        cid = lax.axis_index("c")
        sid = lax.axis_index("s")
        wid = sid * _NC + cid
        r0 = sid * rs
        pltpu.sync_copy(zeros_hbm.at[pl.ds(r0, rs)], acc.at[pl.ds(r0, rs)])
        if tail:
            @pl.when(sid == _NS - 1)
            def _():
                pltpu.sync_copy(zeros_hbm.at[pl.ds(rs * _NS, tail)],
                                acc.at[pl.ds(rs * _NS, tail)])
        plsc.subcore_barrier()

        def fetch(chunk, b):
            pltpu.sync_copy(e3_hbm.at[chunk], sd[b])
            pltpu.async_copy(x_hbm.at[sd[b].at[0]], rb[b], gs[b])

        for b in range(_R - 1):
            fetch(wid + b * _NW, b)

        def body(j, carry):
            for k in range(_R):
                s = _R * j + k
                q = (k + _R - 1) % _R
                pltpu.make_async_copy(x_hbm.at[sd[k].at[0]], rb[k],
                                      gs[k]).wait()
                pltpu.async_copy(rb[k], acc.at[sd[k].at[1]], ss[k], add=True)

                @pl.when(s >= 1)
                def _():
                    pltpu.make_async_copy(rb[q], acc.at[sd[q].at[1]],
                                          ss[q]).wait()

                @pl.when(s + _R - 1 < nbase)
                def _():
                    fetch(wid + (s + _R - 1) * _NW, q)
            return carry

        lax.fori_loop(0, nbase // _R, body, 0)
        pltpu.make_async_copy(rb[_R - 1], acc.at[sd[_R - 1].at[1]],
                              ss[_R - 1]).wait()

        if extra:
            @pl.when(wid < extra)
            def _():
                pltpu.sync_copy(e3_hbm.at[nbase * _NW + wid], sd[0])
                pltpu.async_copy(x_hbm.at[sd[0].at[0]], rb[0], gs[0]).wait()
                pltpu.sync_copy(rb[0], acc.at[sd[0].at[1]], add=True)

        plsc.subcore_barrier()
        pltpu.sync_copy(acc.at[pl.ds(r0, rs)], out_hbm.at[cid, pl.ds(r0, rs)])
        if tail:
            @pl.when(sid == _NS - 1)
            def _():
                pltpu.sync_copy(acc.at[pl.ds(rs * _NS, tail)],
                                out_hbm.at[cid, pl.ds(rs * _NS, tail)])

    return prop


def _make_deg(N, E, W=128):
    n_chunks = E // _C
    rs = (N // _NS) // 8 * 8
    tail = N - rs * _NS
    nbase, extra = divmod(n_chunks, _NW)
    assert nbase % 2 == 0 and nbase >= 4
    mesh = plsc.VectorSubcoreMesh(core_axis_name="c", subcore_axis_name="s")

    @functools.partial(
        pl.kernel,
        mesh=mesh,
        out_type=jax.ShapeDtypeStruct((_NC, N, W), jnp.float32),
        scratch_types=[
            pltpu.VMEM((_C,), jnp.int32),
            pltpu.VMEM((_C,), jnp.int32),
            pltpu.VMEM((_C, W), jnp.float32),
            pltpu.VMEM_SHARED((N, W), jnp.float32),
            pltpu.SemaphoreType.DMA,
            pltpu.SemaphoreType.DMA,
        ],
    )
    def deg(src_hbm, ones_hbm, zeros_hbm, out_hbm,
            idx0, idx1, ones_v, acc, ss0, ss1):
        cid = lax.axis_index("c")
        sid = lax.axis_index("s")
        wid = sid * _NC + cid
        r0 = sid * rs
        pltpu.sync_copy(zeros_hbm.at[pl.ds(r0, rs)], acc.at[pl.ds(r0, rs)])
        if tail:
            @pl.when(sid == _NS - 1)
            def _():
                pltpu.sync_copy(zeros_hbm.at[pl.ds(rs * _NS, tail)],
                                acc.at[pl.ds(rs * _NS, tail)])
        pltpu.sync_copy(ones_hbm, ones_v)
        plsc.subcore_barrier()

        pltpu.sync_copy(src_hbm.at[pl.ds(wid * _C, _C)], idx0)

        def body(j, carry):
            pltpu.async_copy(ones_v, acc.at[idx0], ss0, add=True)

            @pl.when(j >= 1)
            def _():
                pltpu.make_async_copy(ones_v, acc.at[idx1], ss1).wait()

            pltpu.sync_copy(
                src_hbm.at[pl.ds((wid + (2 * j + 1) * _NW) * _C, _C)], idx1)
            pltpu.async_copy(ones_v, acc.at[idx1], ss1, add=True)

            @pl.when(j < nbase // 2 - 1)
            def _():
                pltpu.make_async_copy(ones_v, acc.at[idx0], ss0).wait()
                pltpu.sync_copy(
                    src_hbm.at[pl.ds((wid + (2 * j + 2) * _NW) * _C, _C)],
                    idx0)
            return carry

        lax.fori_loop(0, nbase // 2, body, 0)
        pltpu.make_async_copy(ones_v, acc.at[idx0], ss0).wait()
        pltpu.make_async_copy(ones_v, acc.at[idx1], ss1).wait()

        if extra:
            @pl.when(wid < extra)
            def _():
                pltpu.sync_copy(
                    src_hbm.at[pl.ds((nbase * _NW + wid) * _C, _C)], idx0)
                pltpu.sync_copy(ones_v, acc.at[idx0], add=True)

        plsc.subcore_barrier()
        pltpu.sync_copy(acc.at[pl.ds(r0, rs)], out_hbm.at[cid, pl.ds(r0, rs)])
        if tail:
            @pl.when(sid == _NS - 1)
            def _():
                pltpu.sync_copy(acc.at[pl.ds(rs * _NS, tail)],
                                out_hbm.at[cid, pl.ds(rs * _NS, tail)])

    return deg


_B = 1000


def _tc_pre(degp, x):
    N, D = x.shape

    def body(deg_ref, x_ref, o_ref, dn_ref):
        deg = deg_ref[0, :, 0:1] + deg_ref[1, :, 0:1]
        dinv = jnp.where(deg > 0, lax.rsqrt(jnp.maximum(deg, 1e-12)), 0.0)
        o_ref[...] = dinv * x_ref[...]
        dn_ref[...] = jnp.broadcast_to(dinv, (dinv.shape[0], 8))

    return pl.pallas_call(
        body,
        grid=(N // _B,),
        in_specs=[
            pl.BlockSpec((2, _B, 128), lambda i: (0, i, 0)),
            pl.BlockSpec((_B, D), lambda i: (i, 0)),
        ],
        out_specs=[
            pl.BlockSpec((_B, D), lambda i: (i, 0)),
            pl.BlockSpec((_B, 8), lambda i: (i, 0)),
        ],
        out_shape=[
            jax.ShapeDtypeStruct((N, D), jnp.float32),
            jax.ShapeDtypeStruct((N, 8), jnp.float32),
        ],
    )(degp, x)


def _tc_mid_a(dn, s1):
    N, D = s1.shape[1:]

    def body(dn_ref, s_ref, txp_ref):
        dinv = dn_ref[:, 0:1]
        txp_ref[...] = -(dinv * dinv) * (s_ref[0] + s_ref[1])

    return pl.pallas_call(
        body,
        grid=(N // _B,),
        in_specs=[
            pl.BlockSpec((_B, 8), lambda i: (i, 0)),
            pl.BlockSpec((2, _B, D), lambda i: (0, i, 0)),
        ],
        out_specs=pl.BlockSpec((_B, D), lambda i: (i, 0)),
        out_shape=jax.ShapeDtypeStruct((N, D), jnp.float32),
    )(dn, s1)


def _tc_mid_b(dn, s1, x, W):
    N, D = x.shape

    def body(dn_ref, s_ref, x_ref, w_ref, acc_ref):
        dinv = dn_ref[:, 0:1]
        tx1 = -dinv * (s_ref[0] + s_ref[1])
        acc_ref[...] = (
            jnp.dot(x_ref[...], w_ref[0], preferred_element_type=jnp.float32)
            + jnp.dot(tx1, w_ref[1], preferred_element_type=jnp.float32))

    return pl.pallas_call(
        body,
        grid=(N // _B,),
        in_specs=[
            pl.BlockSpec((_B, 8), lambda i: (i, 0)),
            pl.BlockSpec((2, _B, D), lambda i: (0, i, 0)),
            pl.BlockSpec((_B, D), lambda i: (i, 0)),
            pl.BlockSpec((3, D, D), lambda i: (0, 0, 0)),
        ],
        out_specs=pl.BlockSpec((_B, D), lambda i: (i, 0)),
        out_shape=jax.ShapeDtypeStruct((N, D), jnp.float32),
    )(dn, s1, x, W)


def _tc_fin(dn, s2, x, acc, W, b, relu):
    N, D = x.shape

    def body(dn_ref, s_ref, x_ref, acc_ref, w_ref, b_ref, *outs):
        dinv = dn_ref[:, 0:1]
        tx2 = -2.0 * dinv * (s_ref[0] + s_ref[1]) - x_ref[...]
        o = (acc_ref[...]
             + jnp.dot(tx2, w_ref[2], preferred_element_type=jnp.float32)
             + b_ref[...])
        if relu:
            h = jnp.maximum(o, 0.0)
            outs[0][...] = h
            outs[1][...] = dinv * h
        else:
            outs[0][...] = o

    n_out = 2 if relu else 1
    return pl.pallas_call(
        body,
        grid=(N // _B,),
        in_specs=[
            pl.BlockSpec((_B, 8), lambda i: (i, 0)),
            pl.BlockSpec((2, _B, D), lambda i: (0, i, 0)),
            pl.BlockSpec((_B, D), lambda i: (i, 0)),
            pl.BlockSpec((_B, D), lambda i: (i, 0)),
            pl.BlockSpec((3, D, D), lambda i: (0, 0, 0)),
            pl.BlockSpec((1, D), lambda i: (0, 0)),
        ],
        out_specs=[pl.BlockSpec((_B, D), lambda i: (i, 0))] * n_out,
        out_shape=[jax.ShapeDtypeStruct((N, D), jnp.float32)] * n_out,
    )(dn, s2, x, acc, W, b)


def kernel(x, edge_index, W1, b1, W2, b2):
    N, D = x.shape
    E = edge_index.shape[1]
    src = edge_index[0]
    e3 = edge_index.reshape(2, E // _C, _C).transpose(1, 0, 2)

    zeros_nd = jnp.zeros((N, D), jnp.float32)
    ones_cd = jnp.ones((_C, D), jnp.float32)
    b1r = b1.reshape(1, D)
    b2r = b2.reshape(1, D)

    prop = _make_prop(N, D, E)
    degf = _make_deg(N, E)

    degp = degf(src, ones_cd, zeros_nd)

    xp, dn = _tc_pre(degp, x)
    s1 = prop(xp, e3, zeros_nd)
    tx1p = _tc_mid_a(dn, s1)
    s2 = prop(tx1p, e3, zeros_nd)
    acc1 = _tc_mid_b(dn, s1, x, W1)
    h, hp = _tc_fin(dn, s2, x, acc1, W1, b1r, relu=True)

    s3 = prop(hp, e3, zeros_nd)
    tx1p2 = _tc_mid_a(dn, s3)
    s4 = prop(tx1p2, e3, zeros_nd)
    acc2 = _tc_mid_b(dn, s3, h, W2)
    (out,) = _tc_fin(dn, s4, h, acc2, W2, b2r, relu=False)
    return out


# speedup vs baseline: 1.0729x; 1.0159x over previous
import functools

import jax
import jax.numpy as jnp
from jax import lax
from jax.experimental import pallas as pl
from jax.experimental.pallas import tpu as pltpu
from jax.experimental.pallas import tpu_sc as plsc

_INFO = plsc.get_sparse_core_info()
_NC = _INFO.num_cores
_NS = _INFO.num_subcores
_NW = _NC * _NS
_C = 128


_R = 3


def _make_prop(N, D, E):
    n_chunks = E // _C
    assert n_chunks * _C == E
    rs = (N // _NS) // 8 * 8
    tail = N - rs * _NS
    nbase, extra = divmod(n_chunks, _NW)
    assert nbase % _R == 0 and nbase >= 2 * _R
    mesh = plsc.VectorSubcoreMesh(core_axis_name="c", subcore_axis_name="s")

    @functools.partial(
        pl.kernel,
        mesh=mesh,
        out_type=jax.ShapeDtypeStruct((_NC, N, D), jnp.float32),
        scratch_types=(
            [pltpu.VMEM((2, _C), jnp.int32)] * _R
            + [pltpu.VMEM((_C, D), jnp.float32)] * _R
            + [pltpu.VMEM_SHARED((N, D), jnp.float32)]
            + [pltpu.SemaphoreType.DMA] * (2 * _R)
        ),
    )
    def prop(x_hbm, e3_hbm, zeros_hbm, out_hbm, *scr):
        sd = scr[:_R]
        rb = scr[_R:2 * _R]
        acc = scr[2 * _R]
        gs = scr[2 * _R + 1:3 * _R + 1]
        ss = scr[3 * _R + 1:4 * _R + 1]
        cid = lax.axis_index("c")
        sid = lax.axis_index("s")
        wid = sid * _NC + cid
        r0 = sid * rs
        pltpu.sync_copy(zeros_hbm.at[pl.ds(r0, rs)], acc.at[pl.ds(r0, rs)])
        if tail:
            @pl.when(sid == _NS - 1)
            def _():
                pltpu.sync_copy(zeros_hbm.at[pl.ds(rs * _NS, tail)],
                                acc.at[pl.ds(rs * _NS, tail)])
        plsc.subcore_barrier()

        def fetch(chunk, b):
            pltpu.sync_copy(e3_hbm.at[chunk], sd[b])
            pltpu.async_copy(x_hbm.at[sd[b].at[0]], rb[b], gs[b])

        for b in range(_R - 1):
            fetch(wid + b * _NW, b)

        def body(j, carry):
            for k in range(_R):
                s = _R * j + k
                q = (k + _R - 1) % _R
                pltpu.make_async_copy(x_hbm.at[sd[k].at[0]], rb[k],
                                      gs[k]).wait()
                pltpu.async_copy(rb[k], acc.at[sd[k].at[1]], ss[k], add=True)

                @pl.when(s >= 1)
                def _():
                    pltpu.make_async_copy(rb[q], acc.at[sd[q].at[1]],
                                          ss[q]).wait()

                @pl.when(s + _R - 1 < nbase)
                def _():
                    fetch(wid + (s + _R - 1) * _NW, q)
            return carry

        lax.fori_loop(0, nbase // _R, body, 0)
        pltpu.make_async_copy(rb[_R - 1], acc.at[sd[_R - 1].at[1]],
                              ss[_R - 1]).wait()

        if extra:
            @pl.when(wid < extra)
            def _():
                pltpu.sync_copy(e3_hbm.at[nbase * _NW + wid], sd[0])
                pltpu.async_copy(x_hbm.at[sd[0].at[0]], rb[0], gs[0]).wait()
                pltpu.sync_copy(rb[0], acc.at[sd[0].at[1]], add=True)

        plsc.subcore_barrier()
        pltpu.sync_copy(acc.at[pl.ds(r0, rs)], out_hbm.at[cid, pl.ds(r0, rs)])
        if tail:
            @pl.when(sid == _NS - 1)
            def _():
                pltpu.sync_copy(acc.at[pl.ds(rs * _NS, tail)],
                                out_hbm.at[cid, pl.ds(rs * _NS, tail)])

    return prop


def _make_deg(N, E, W=128):
    n_chunks = E // _C
    rs = (N // _NS) // 8 * 8
    tail = N - rs * _NS
    nbase, extra = divmod(n_chunks, _NW)
    assert nbase % 2 == 0 and nbase >= 4
    mesh = plsc.VectorSubcoreMesh(core_axis_name="c", subcore_axis_name="s")

    @functools.partial(
        pl.kernel,
        mesh=mesh,
        out_type=jax.ShapeDtypeStruct((_NC, N, W), jnp.float32),
        scratch_types=[
            pltpu.VMEM((_C,), jnp.int32),
            pltpu.VMEM((_C,), jnp.int32),
            pltpu.VMEM((_C, W), jnp.float32),
            pltpu.VMEM_SHARED((N, W), jnp.float32),
            pltpu.SemaphoreType.DMA,
            pltpu.SemaphoreType.DMA,
        ],
    )
    def deg(src_hbm, ones_hbm, zeros_hbm, out_hbm,
            idx0, idx1, ones_v, acc, ss0, ss1):
        cid = lax.axis_index("c")
        sid = lax.axis_index("s")
        wid = sid * _NC + cid
        r0 = sid * rs
        pltpu.sync_copy(zeros_hbm.at[pl.ds(r0, rs)], acc.at[pl.ds(r0, rs)])
        if tail:
            @pl.when(sid == _NS - 1)
            def _():
                pltpu.sync_copy(zeros_hbm.at[pl.ds(rs * _NS, tail)],
                                acc.at[pl.ds(rs * _NS, tail)])
        pltpu.sync_copy(ones_hbm, ones_v)
        plsc.subcore_barrier()

        pltpu.sync_copy(src_hbm.at[pl.ds(wid * _C, _C)], idx0)

        def body(j, carry):
            pltpu.async_copy(ones_v, acc.at[idx0], ss0, add=True)

            @pl.when(j >= 1)
            def _():
                pltpu.make_async_copy(ones_v, acc.at[idx1], ss1).wait()

            pltpu.sync_copy(
                src_hbm.at[pl.ds((wid + (2 * j + 1) * _NW) * _C, _C)], idx1)
            pltpu.async_copy(ones_v, acc.at[idx1], ss1, add=True)

            @pl.when(j < nbase // 2 - 1)
            def _():
                pltpu.make_async_copy(ones_v, acc.at[idx0], ss0).wait()
                pltpu.sync_copy(
                    src_hbm.at[pl.ds((wid + (2 * j + 2) * _NW) * _C, _C)],
                    idx0)
            return carry

        lax.fori_loop(0, nbase // 2, body, 0)
        pltpu.make_async_copy(ones_v, acc.at[idx0], ss0).wait()
        pltpu.make_async_copy(ones_v, acc.at[idx1], ss1).wait()

        if extra:
            @pl.when(wid < extra)
            def _():
                pltpu.sync_copy(
                    src_hbm.at[pl.ds((nbase * _NW + wid) * _C, _C)], idx0)
                pltpu.sync_copy(ones_v, acc.at[idx0], add=True)

        plsc.subcore_barrier()
        pltpu.sync_copy(acc.at[pl.ds(r0, rs)], out_hbm.at[cid, pl.ds(r0, rs)])
        if tail:
            @pl.when(sid == _NS - 1)
            def _():
                pltpu.sync_copy(acc.at[pl.ds(rs * _NS, tail)],
                                out_hbm.at[cid, pl.ds(rs * _NS, tail)])

    return deg


_B = 2000


def _tc_pre(degp, x):
    N, D = x.shape

    def body(deg_ref, x_ref, o_ref, dn_ref):
        deg = deg_ref[0, :, 0:1] + deg_ref[1, :, 0:1]
        dinv = jnp.where(deg > 0, lax.rsqrt(jnp.maximum(deg, 1e-12)), 0.0)
        o_ref[...] = dinv * x_ref[...]
        dn_ref[...] = jnp.broadcast_to(dinv, (dinv.shape[0], 8))

    return pl.pallas_call(
        body,
        grid=(N // _B,),
        in_specs=[
            pl.BlockSpec((2, _B, 128), lambda i: (0, i, 0)),
            pl.BlockSpec((_B, D), lambda i: (i, 0)),
        ],
        out_specs=[
            pl.BlockSpec((_B, D), lambda i: (i, 0)),
            pl.BlockSpec((_B, 8), lambda i: (i, 0)),
        ],
        out_shape=[
            jax.ShapeDtypeStruct((N, D), jnp.float32),
            jax.ShapeDtypeStruct((N, 8), jnp.float32),
        ],
    )(degp, x)


def _tc_mid_a(dn, s1):
    N, D = s1.shape[1:]

    def body(dn_ref, s_ref, txp_ref):
        dinv = dn_ref[:, 0:1]
        txp_ref[...] = -(dinv * dinv) * (s_ref[0] + s_ref[1])

    return pl.pallas_call(
        body,
        grid=(N // _B,),
        in_specs=[
            pl.BlockSpec((_B, 8), lambda i: (i, 0)),
            pl.BlockSpec((2, _B, D), lambda i: (0, i, 0)),
        ],
        out_specs=pl.BlockSpec((_B, D), lambda i: (i, 0)),
        out_shape=jax.ShapeDtypeStruct((N, D), jnp.float32),
    )(dn, s1)


def _tc_mid_b(dn, s1, x, W):
    N, D = x.shape

    def body(dn_ref, s_ref, x_ref, w_ref, acc_ref):
        dinv = dn_ref[:, 0:1]
        tx1 = -dinv * (s_ref[0] + s_ref[1])
        acc_ref[...] = (
            jnp.dot(x_ref[...], w_ref[0], preferred_element_type=jnp.float32)
            + jnp.dot(tx1, w_ref[1], preferred_element_type=jnp.float32))

    return pl.pallas_call(
        body,
        grid=(N // _B,),
        in_specs=[
            pl.BlockSpec((_B, 8), lambda i: (i, 0)),
            pl.BlockSpec((2, _B, D), lambda i: (0, i, 0)),
            pl.BlockSpec((_B, D), lambda i: (i, 0)),
            pl.BlockSpec((3, D, D), lambda i: (0, 0, 0)),
        ],
        out_specs=pl.BlockSpec((_B, D), lambda i: (i, 0)),
        out_shape=jax.ShapeDtypeStruct((N, D), jnp.float32),
    )(dn, s1, x, W)


def _tc_fin(dn, s2, x, acc, W, b, relu):
    N, D = x.shape

    def body(dn_ref, s_ref, x_ref, acc_ref, w_ref, b_ref, *outs):
        dinv = dn_ref[:, 0:1]
        tx2 = -2.0 * dinv * (s_ref[0] + s_ref[1]) - x_ref[...]
        o = (acc_ref[...]
             + jnp.dot(tx2, w_ref[2], preferred_element_type=jnp.float32)
             + b_ref[...])
        if relu:
            h = jnp.maximum(o, 0.0)
            outs[0][...] = h
            outs[1][...] = dinv * h
        else:
            outs[0][...] = o

    n_out = 2 if relu else 1
    return pl.pallas_call(
        body,
        grid=(N // _B,),
        in_specs=[
            pl.BlockSpec((_B, 8), lambda i: (i, 0)),
            pl.BlockSpec((2, _B, D), lambda i: (0, i, 0)),
            pl.BlockSpec((_B, D), lambda i: (i, 0)),
            pl.BlockSpec((_B, D), lambda i: (i, 0)),
            pl.BlockSpec((3, D, D), lambda i: (0, 0, 0)),
            pl.BlockSpec((1, D), lambda i: (0, 0)),
        ],
        out_specs=[pl.BlockSpec((_B, D), lambda i: (i, 0))] * n_out,
        out_shape=[jax.ShapeDtypeStruct((N, D), jnp.float32)] * n_out,
    )(dn, s2, x, acc, W, b)


def kernel(x, edge_index, W1, b1, W2, b2):
    N, D = x.shape
    E = edge_index.shape[1]
    src = edge_index[0]
    e3 = edge_index.reshape(2, E // _C, _C).transpose(1, 0, 2)

    zeros_nd = jnp.zeros((N, D), jnp.float32)
    ones_cd = jnp.ones((_C, D), jnp.float32)
    b1r = b1.reshape(1, D)
    b2r = b2.reshape(1, D)

    prop = _make_prop(N, D, E)
    degf = _make_deg(N, E)

    degp = degf(src, ones_cd, zeros_nd)

    xp, dn = _tc_pre(degp, x)
    s1 = prop(xp, e3, zeros_nd)
    tx1p = _tc_mid_a(dn, s1)
    s2 = prop(tx1p, e3, zeros_nd)
    acc1 = _tc_mid_b(dn, s1, x, W1)
    h, hp = _tc_fin(dn, s2, x, acc1, W1, b1r, relu=True)

    s3 = prop(hp, e3, zeros_nd)
    tx1p2 = _tc_mid_a(dn, s3)
    s4 = prop(tx1p2, e3, zeros_nd)
    acc2 = _tc_mid_b(dn, s3, h, W2)
    (out,) = _tc_fin(dn, s4, h, acc2, W2, b2r, relu=False)
    return out


# speedup vs baseline: 1.0774x; 1.0042x over previous
import functools

import jax
import jax.numpy as jnp
from jax import lax
from jax.experimental import pallas as pl
from jax.experimental.pallas import tpu as pltpu
from jax.experimental.pallas import tpu_sc as plsc

_INFO = plsc.get_sparse_core_info()
_NC = _INFO.num_cores
_NS = _INFO.num_subcores
_NW = _NC * _NS
_C = 128


_R = 3


def _make_prop(N, D, E):
    n_chunks = E // _C
    assert n_chunks * _C == E
    rs = (N // _NS) // 8 * 8
    tail = N - rs * _NS
    nbase, extra = divmod(n_chunks, _NW)
    assert nbase % _R == 0 and nbase >= 2 * _R
    mesh = plsc.VectorSubcoreMesh(core_axis_name="c", subcore_axis_name="s")

    @functools.partial(
        pl.kernel,
        mesh=mesh,
        out_type=jax.ShapeDtypeStruct((_NC, N, D), jnp.float32),
        scratch_types=(
            [pltpu.VMEM((2, _C), jnp.int32)] * _R
            + [pltpu.VMEM((_C, D), jnp.float32)] * _R
            + [pltpu.VMEM_SHARED((N, D), jnp.float32)]
            + [pltpu.SemaphoreType.DMA] * (2 * _R)
        ),
    )
    def prop(x_hbm, e3_hbm, zeros_hbm, out_hbm, *scr):
        sd = scr[:_R]
        rb = scr[_R:2 * _R]
        acc = scr[2 * _R]
        gs = scr[2 * _R + 1:3 * _R + 1]
        ss = scr[3 * _R + 1:4 * _R + 1]
        cid = lax.axis_index("c")
        sid = lax.axis_index("s")
        wid = sid * _NC + cid
        r0 = sid * rs
        pltpu.sync_copy(zeros_hbm.at[pl.ds(r0, rs)], acc.at[pl.ds(r0, rs)])
        if tail:
            @pl.when(sid == _NS - 1)
            def _():
                pltpu.sync_copy(zeros_hbm.at[pl.ds(rs * _NS, tail)],
                                acc.at[pl.ds(rs * _NS, tail)])
        plsc.subcore_barrier()

        def fetch(chunk, b):
            pltpu.sync_copy(e3_hbm.at[chunk], sd[b])
            pltpu.async_copy(x_hbm.at[sd[b].at[0]], rb[b], gs[b])

        for b in range(_R - 1):
            fetch(wid + b * _NW, b)

        def body(j, carry):
            for k in range(_R):
                s = _R * j + k
                q = (k + _R - 1) % _R
                pltpu.make_async_copy(x_hbm.at[sd[k].at[0]], rb[k],
                                      gs[k]).wait()
                pltpu.async_copy(rb[k], acc.at[sd[k].at[1]], ss[k], add=True)

                @pl.when(s >= 1)
                def _():
                    pltpu.make_async_copy(rb[q], acc.at[sd[q].at[1]],
                                          ss[q]).wait()

                @pl.when(s + _R - 1 < nbase)
                def _():
                    fetch(wid + (s + _R - 1) * _NW, q)
            return carry

        lax.fori_loop(0, nbase // _R, body, 0)
        pltpu.make_async_copy(rb[_R - 1], acc.at[sd[_R - 1].at[1]],
                              ss[_R - 1]).wait()

        if extra:
            @pl.when(wid < extra)
            def _():
                pltpu.sync_copy(e3_hbm.at[nbase * _NW + wid], sd[0])
                pltpu.async_copy(x_hbm.at[sd[0].at[0]], rb[0], gs[0]).wait()
                pltpu.sync_copy(rb[0], acc.at[sd[0].at[1]], add=True)

        plsc.subcore_barrier()
        pltpu.sync_copy(acc.at[pl.ds(r0, rs)], out_hbm.at[cid, pl.ds(r0, rs)])
        if tail:
            @pl.when(sid == _NS - 1)
            def _():
                pltpu.sync_copy(acc.at[pl.ds(rs * _NS, tail)],
                                out_hbm.at[cid, pl.ds(rs * _NS, tail)])

    return prop


def _make_deg(N, E, W=128):
    n_chunks = E // _C
    rs = (N // _NS) // 8 * 8
    tail = N - rs * _NS
    nbase, extra = divmod(n_chunks, _NW)
    assert nbase % 2 == 0 and nbase >= 4
    mesh = plsc.VectorSubcoreMesh(core_axis_name="c", subcore_axis_name="s")

    @functools.partial(
        pl.kernel,
        mesh=mesh,
        out_type=jax.ShapeDtypeStruct((_NC, N, W), jnp.float32),
        scratch_types=[
            pltpu.VMEM((_C,), jnp.int32),
            pltpu.VMEM((_C,), jnp.int32),
            pltpu.VMEM((_C, W), jnp.float32),
            pltpu.VMEM_SHARED((N, W), jnp.float32),
            pltpu.SemaphoreType.DMA,
            pltpu.SemaphoreType.DMA,
        ],
    )
    def deg(src_hbm, ones_hbm, zeros_hbm, out_hbm,
            idx0, idx1, ones_v, acc, ss0, ss1):
        cid = lax.axis_index("c")
        sid = lax.axis_index("s")
        wid = sid * _NC + cid
        r0 = sid * rs
        pltpu.sync_copy(zeros_hbm.at[pl.ds(r0, rs)], acc.at[pl.ds(r0, rs)])
        if tail:
            @pl.when(sid == _NS - 1)
            def _():
                pltpu.sync_copy(zeros_hbm.at[pl.ds(rs * _NS, tail)],
                                acc.at[pl.ds(rs * _NS, tail)])
        pltpu.sync_copy(ones_hbm, ones_v)
        plsc.subcore_barrier()

        pltpu.sync_copy(src_hbm.at[pl.ds(wid * _C, _C)], idx0)

        def body(j, carry):
            pltpu.async_copy(ones_v, acc.at[idx0], ss0, add=True)

            @pl.when(j >= 1)
            def _():
                pltpu.make_async_copy(ones_v, acc.at[idx1], ss1).wait()

            pltpu.sync_copy(
                src_hbm.at[pl.ds((wid + (2 * j + 1) * _NW) * _C, _C)], idx1)
            pltpu.async_copy(ones_v, acc.at[idx1], ss1, add=True)

            @pl.when(j < nbase // 2 - 1)
            def _():
                pltpu.make_async_copy(ones_v, acc.at[idx0], ss0).wait()
                pltpu.sync_copy(
                    src_hbm.at[pl.ds((wid + (2 * j + 2) * _NW) * _C, _C)],
                    idx0)
            return carry

        lax.fori_loop(0, nbase // 2, body, 0)
        pltpu.make_async_copy(ones_v, acc.at[idx0], ss0).wait()
        pltpu.make_async_copy(ones_v, acc.at[idx1], ss1).wait()

        if extra:
            @pl.when(wid < extra)
            def _():
                pltpu.sync_copy(
                    src_hbm.at[pl.ds((nbase * _NW + wid) * _C, _C)], idx0)
                pltpu.sync_copy(ones_v, acc.at[idx0], add=True)

        plsc.subcore_barrier()
        pltpu.sync_copy(acc.at[pl.ds(r0, rs)], out_hbm.at[cid, pl.ds(r0, rs)])
        if tail:
            @pl.when(sid == _NS - 1)
            def _():
                pltpu.sync_copy(acc.at[pl.ds(rs * _NS, tail)],
                                out_hbm.at[cid, pl.ds(rs * _NS, tail)])

    return deg


_B = 5000


def _tc_pre(degp, x):
    N, D = x.shape

    def body(deg_ref, x_ref, o_ref, dn_ref):
        deg = deg_ref[0, :, 0:1] + deg_ref[1, :, 0:1]
        dinv = jnp.where(deg > 0, lax.rsqrt(jnp.maximum(deg, 1e-12)), 0.0)
        o_ref[...] = dinv * x_ref[...]
        dn_ref[...] = jnp.broadcast_to(dinv, (dinv.shape[0], 8))

    return pl.pallas_call(
        body,
        grid=(N // _B,),
        in_specs=[
            pl.BlockSpec((2, _B, 128), lambda i: (0, i, 0)),
            pl.BlockSpec((_B, D), lambda i: (i, 0)),
        ],
        out_specs=[
            pl.BlockSpec((_B, D), lambda i: (i, 0)),
            pl.BlockSpec((_B, 8), lambda i: (i, 0)),
        ],
        out_shape=[
            jax.ShapeDtypeStruct((N, D), jnp.float32),
            jax.ShapeDtypeStruct((N, 8), jnp.float32),
        ],
    )(degp, x)


def _tc_mid_a(dn, s1):
    N, D = s1.shape[1:]

    def body(dn_ref, s_ref, txp_ref):
        dinv = dn_ref[:, 0:1]
        txp_ref[...] = -(dinv * dinv) * (s_ref[0] + s_ref[1])

    return pl.pallas_call(
        body,
        grid=(N // _B,),
        in_specs=[
            pl.BlockSpec((_B, 8), lambda i: (i, 0)),
            pl.BlockSpec((2, _B, D), lambda i: (0, i, 0)),
        ],
        out_specs=pl.BlockSpec((_B, D), lambda i: (i, 0)),
        out_shape=jax.ShapeDtypeStruct((N, D), jnp.float32),
    )(dn, s1)


def _tc_mid_b(dn, s1, x, W):
    N, D = x.shape

    def body(dn_ref, s_ref, x_ref, w_ref, acc_ref):
        dinv = dn_ref[:, 0:1]
        tx1 = -dinv * (s_ref[0] + s_ref[1])
        acc_ref[...] = (
            jnp.dot(x_ref[...], w_ref[0], preferred_element_type=jnp.float32)
            + jnp.dot(tx1, w_ref[1], preferred_element_type=jnp.float32))

    return pl.pallas_call(
        body,
        grid=(N // _B,),
        in_specs=[
            pl.BlockSpec((_B, 8), lambda i: (i, 0)),
            pl.BlockSpec((2, _B, D), lambda i: (0, i, 0)),
            pl.BlockSpec((_B, D), lambda i: (i, 0)),
            pl.BlockSpec((3, D, D), lambda i: (0, 0, 0)),
        ],
        out_specs=pl.BlockSpec((_B, D), lambda i: (i, 0)),
        out_shape=jax.ShapeDtypeStruct((N, D), jnp.float32),
    )(dn, s1, x, W)


def _tc_fin(dn, s2, x, acc, W, b, relu):
    N, D = x.shape

    def body(dn_ref, s_ref, x_ref, acc_ref, w_ref, b_ref, *outs):
        dinv = dn_ref[:, 0:1]
        tx2 = -2.0 * dinv * (s_ref[0] + s_ref[1]) - x_ref[...]
        o = (acc_ref[...]
             + jnp.dot(tx2, w_ref[2], preferred_element_type=jnp.float32)
             + b_ref[...])
        if relu:
            h = jnp.maximum(o, 0.0)
            outs[0][...] = h
            outs[1][...] = dinv * h
        else:
            outs[0][...] = o

    n_out = 2 if relu else 1
    return pl.pallas_call(
        body,
        grid=(N // _B,),
        in_specs=[
            pl.BlockSpec((_B, 8), lambda i: (i, 0)),
            pl.BlockSpec((2, _B, D), lambda i: (0, i, 0)),
            pl.BlockSpec((_B, D), lambda i: (i, 0)),
            pl.BlockSpec((_B, D), lambda i: (i, 0)),
            pl.BlockSpec((3, D, D), lambda i: (0, 0, 0)),
            pl.BlockSpec((1, D), lambda i: (0, 0)),
        ],
        out_specs=[pl.BlockSpec((_B, D), lambda i: (i, 0))] * n_out,
        out_shape=[jax.ShapeDtypeStruct((N, D), jnp.float32)] * n_out,
    )(dn, s2, x, acc, W, b)


def kernel(x, edge_index, W1, b1, W2, b2):
    N, D = x.shape
    E = edge_index.shape[1]
    src = edge_index[0]
    e3 = edge_index.reshape(2, E // _C, _C).transpose(1, 0, 2)

    zeros_nd = jnp.zeros((N, D), jnp.float32)
    ones_cd = jnp.ones((_C, D), jnp.float32)
    b1r = b1.reshape(1, D)
    b2r = b2.reshape(1, D)

    prop = _make_prop(N, D, E)
    degf = _make_deg(N, E)

    degp = degf(src, ones_cd, zeros_nd)

    xp, dn = _tc_pre(degp, x)
    s1 = prop(xp, e3, zeros_nd)
    tx1p = _tc_mid_a(dn, s1)
    s2 = prop(tx1p, e3, zeros_nd)
    acc1 = _tc_mid_b(dn, s1, x, W1)
    h, hp = _tc_fin(dn, s2, x, acc1, W1, b1r, relu=True)

    s3 = prop(hp, e3, zeros_nd)
    tx1p2 = _tc_mid_a(dn, s3)
    s4 = prop(tx1p2, e3, zeros_nd)
    acc2 = _tc_mid_b(dn, s3, h, W2)
    (out,) = _tc_fin(dn, s4, h, acc2, W2, b2r, relu=False)
    return out
